# baseline (speedup 1.0000x reference)
import functools

import jax
import jax.numpy as jnp
from jax import lax
from jax.experimental import pallas as pl
from jax.experimental.pallas import tpu as pltpu
from jax.experimental.pallas import tpu_sc as plsc


def _make_sc_gather(entity_dim: int, batch: int):
    info = plsc.get_sparse_core_info()
    nw = info.num_cores * info.num_subcores
    assert batch % nw == 0
    b_per_w = batch // nw
    lanes = info.num_lanes

    mesh = plsc.VectorSubcoreMesh(core_axis_name="c", subcore_axis_name="s")

    @functools.partial(
        pl.kernel,
        mesh=mesh,
        out_type=jax.ShapeDtypeStruct((batch, 128), jnp.float32),
        scratch_types=[
            pltpu.VMEM((batch,), jnp.int32),
            pltpu.VMEM((lanes, entity_dim, 128), jnp.float32),
            pltpu.VMEM((b_per_w, 128), jnp.float32),
            pltpu.SemaphoreType.DMA,
        ],
        compiler_params=pltpu.CompilerParams(needs_layout_passes=False),
    )
    def gather_kernel(table_hbm, idx_hbm, out_hbm, idx_v, blk_v, out_v, sem):
        wid = lax.axis_index("s") * info.num_cores + lax.axis_index("c")
        base = wid * b_per_w
        pltpu.sync_copy(idx_hbm, idx_v)
        for w in range(b_per_w // lanes):
            iv = idx_v[pl.ds(base + w * lanes, lanes)]
            ctv = (iv >> 7) << 7
            copies = []
            for jj in range(lanes):
                off = pl.multiple_of(ctv[jj], 128)
                for r in range(entity_dim // 8):
                    copies.append(
                        pltpu.async_copy(
                            table_hbm.at[pl.ds(r * 8, 8), pl.ds(off, 128)],
                            blk_v.at[jj, pl.ds(r * 8, 8)],
                            sem,
                        )
                    )
            for c in copies:
                c.wait()
            cov = iv & 127
            jv_local = lax.iota(jnp.int32, lanes)
            jv_out = jv_local + w * lanes
            for k in range(entity_dim):
                kv = jnp.full((lanes,), k, jnp.int32)
                vals = plsc.load_gather(blk_v, [jv_local, kv, cov])
                plsc.store_scatter(out_v, [jv_out, kv], vals)
        pltpu.sync_copy(out_v, out_hbm.at[pl.ds(base, b_per_w)])

    return gather_kernel


def _project_body(g_ref, w_ref, b_ref, o_ref):
    o_ref[...] = (
        lax.dot_general(
            w_ref[...],
            g_ref[:, :32],
            (((0,), (1,)), ((), ())),
            preferred_element_type=jnp.float32,
        )
        + b_ref[...]
    )


def kernel(x, entity_bank, W, b):
    batch = x.shape[0]
    num_entities, entity_dim = entity_bank.shape
    out_dim = W.shape[1]

    idx = x.reshape(batch).astype(jnp.int32)
    gathered = _make_sc_gather(entity_dim, batch)(entity_bank.T, idx)

    out_t = pl.pallas_call(
        _project_body,
        out_shape=jax.ShapeDtypeStruct((out_dim, batch), jnp.float32),
    )(gathered, W, b.reshape(out_dim, 1))
    return out_t.T


# speedup vs baseline: 1.0452x; 1.0452x over previous
import functools

import jax
import jax.numpy as jnp
from jax import lax
from jax.experimental import pallas as pl
from jax.experimental.pallas import tpu as pltpu
from jax.experimental.pallas import tpu_sc as plsc


def _make_sc_gather(entity_dim: int, batch: int):
    info = plsc.get_sparse_core_info()
    nw = info.num_cores * info.num_subcores
    assert batch % nw == 0
    b_per_w = batch // nw
    lanes = info.num_lanes

    mesh = plsc.VectorSubcoreMesh(core_axis_name="c", subcore_axis_name="s")

    @functools.partial(
        pl.kernel,
        mesh=mesh,
        out_type=jax.ShapeDtypeStruct((batch, 128), jnp.float32),
        scratch_types=[
            pltpu.VMEM((batch,), jnp.int32),
            pltpu.VMEM((lanes, entity_dim, 128), jnp.float32),
            pltpu.VMEM((b_per_w, 128), jnp.float32),
            pltpu.SemaphoreType.DMA,
        ],
        compiler_params=pltpu.CompilerParams(needs_layout_passes=False),
    )
    def gather_kernel(table_hbm, idx_hbm, out_hbm, idx_v, blk_v, out_v, sem):
        wid = lax.axis_index("s") * info.num_cores + lax.axis_index("c")
        base = wid * b_per_w
        pltpu.sync_copy(idx_hbm, idx_v)
        for w in range(b_per_w // lanes):
            iv = idx_v[pl.ds(base + w * lanes, lanes)]
            ctv = (iv >> 7) << 7
            copies = []
            for jj in range(lanes):
                off = pl.multiple_of(ctv[jj], 128)
                copies.append(
                    pltpu.async_copy(
                        table_hbm.at[:, pl.ds(off, 128)], blk_v.at[jj], sem
                    )
                )
            for c in copies:
                c.wait()
            cov = iv & 127
            jv_local = lax.iota(jnp.int32, lanes)
            jv_out = jv_local + w * lanes
            for k in range(entity_dim):
                kv = jnp.full((lanes,), k, jnp.int32)
                vals = plsc.load_gather(blk_v, [jv_local, kv, cov])
                plsc.store_scatter(out_v, [jv_out, kv], vals)
        pltpu.sync_copy(out_v, out_hbm.at[pl.ds(base, b_per_w)])

    return gather_kernel


def _project_body(g_ref, w_ref, b_ref, o_ref):
    o_ref[...] = (
        lax.dot_general(
            w_ref[...],
            g_ref[:, :32],
            (((0,), (1,)), ((), ())),
            preferred_element_type=jnp.float32,
        )
        + b_ref[...]
    )


def kernel(x, entity_bank, W, b):
    batch = x.shape[0]
    num_entities, entity_dim = entity_bank.shape
    out_dim = W.shape[1]

    idx = x.reshape(batch).astype(jnp.int32)
    gathered = _make_sc_gather(entity_dim, batch)(entity_bank.T, idx)

    out_t = pl.pallas_call(
        _project_body,
        out_shape=jax.ShapeDtypeStruct((out_dim, batch), jnp.float32),
    )(gathered, W, b.reshape(out_dim, 1))
    return out_t.T
